# baseline (speedup 1.0000x reference)
import functools

import jax
import jax.numpy as jnp
from jax import lax
from jax.experimental import pallas as pl
from jax.experimental.pallas import tpu as pltpu
from jax.experimental.pallas import tpu_sc as plsc

_N = 50000
_E = 800000
_NC, _NS = 2, 16
_NW = _NC * _NS
_ECH = _E // _NW
_NVEC = (_ECH + 15) // 16
_EBUF = _NVEC * 16
_NR = 8
_RR = 6256
_STR = 392
_ACCR = _NS * _STR
_DUMP = 6264
_CH = 256
_NCH = 100
_CAP = _NCH * _CH

_mesh = plsc.VectorSubcoreMesh(core_axis_name="c", subcore_axis_name="s")


def _wid():
    return lax.axis_index("s") * _NC + lax.axis_index("c")


@functools.partial(
    pl.kernel,
    out_type=jax.ShapeDtypeStruct((_NW, 1, _N), jnp.float32),
    mesh=_mesh,
    scratch_types=[
        pltpu.VMEM((_EBUF,), jnp.int32),
        pltpu.VMEM((_N,), jnp.float32),
    ],
    compiler_params=pltpu.CompilerParams(needs_layout_passes=False),
)
def _hist_kernel(d_hbm, hist_hbm, dst_v, hist_v):
    w = _wid()
    base = w * _ECH
    pltpu.sync_copy(d_hbm.at[pl.ds(base, _ECH)], dst_v.at[pl.ds(0, _ECH)])

    def zero(i, carry):
        hist_v[pl.ds(i * 16, 16)] = jnp.zeros((16,), jnp.float32)
        return carry

    lax.fori_loop(0, _N // 16, zero, 0)

    iota = lax.iota(jnp.int32, 16)
    ones = jnp.ones((16,), jnp.float32)

    def body(i, carry):
        d16 = dst_v[pl.ds(i * 16, 16)]
        m = (i * 16 + iota) < _ECH
        plsc.addupdate_scatter(hist_v, [d16], ones, mask=m)
        return carry

    lax.fori_loop(0, _NVEC, body, 0)
    pltpu.sync_copy(hist_v, hist_hbm.at[w, 0])


@functools.partial(
    pl.kernel,
    out_type=(
        jax.ShapeDtypeStruct((_NR, _NW, 1, _CAP), jnp.int32),
        jax.ShapeDtypeStruct((_NR, _NW, 1, _CAP), jnp.int32),
        jax.ShapeDtypeStruct((_NR, _NW, 1, 128), jnp.int32),
    ),
    mesh=_mesh,
    scratch_types=[
        pltpu.VMEM((_EBUF,), jnp.int32),
        pltpu.VMEM((_EBUF,), jnp.int32),
        pltpu.VMEM((_CAP,), jnp.int32),
        pltpu.VMEM((_CAP,), jnp.int32),
        pltpu.VMEM((128,), jnp.int32),
    ],
    compiler_params=pltpu.CompilerParams(needs_layout_passes=False),
)
def _lists_kernel(s_hbm, d_hbm, srcl_hbm, dstl_hbm, cnt_hbm,
                  src_v, dst_v, srcl_v, dstl_v, ncb_v):
    w = _wid()
    base = w * _ECH
    pltpu.sync_copy(s_hbm.at[pl.ds(base, _ECH)], src_v.at[pl.ds(0, _ECH)])
    pltpu.sync_copy(d_hbm.at[pl.ds(base, _ECH)], dst_v.at[pl.ds(0, _ECH)])
    iota = lax.iota(jnp.int32, 16)
    zs16 = jnp.zeros((16,), jnp.int32)
    dump16 = jnp.full((16,), _DUMP, jnp.int32)

    for r in range(_NR):
        lo = r * _RR

        def body(i, off):
            s16 = src_v[pl.ds(i * 16, 16)]
            d16 = dst_v[pl.ds(i * 16, 16)]
            m = ((i * 16 + iota) < _ECH) & (d16 >= lo) & (d16 < lo + _RR)
            inc = jnp.cumsum(m.astype(jnp.int32))
            pos = off + inc - 1
            plsc.store_scatter(srcl_v, [pos], s16, mask=m)
            plsc.store_scatter(dstl_v, [pos], d16 - lo, mask=m)
            return off + jnp.max(inc)

        off = lax.fori_loop(0, _NVEC, body, jnp.int32(0))

        def fill(t, c):
            srcl_v[pl.ds(off + t * 16, 16)] = zs16
            dstl_v[pl.ds(off + t * 16, 16)] = dump16
            return c

        lax.fori_loop(0, 2 * _CH // 16, fill, 0)
        nch = ((off + 2 * _CH - 1) // (2 * _CH)) * 2

        def setnc(i, c):
            ncb_v[pl.ds(i * 16, 16)] = jnp.broadcast_to(nch, (16,)).astype(
                jnp.int32)
            return c

        lax.fori_loop(0, 8, setnc, 0)
        pltpu.sync_copy(srcl_v, srcl_hbm.at[r, w, 0])
        pltpu.sync_copy(dstl_v, dstl_hbm.at[r, w, 0])
        pltpu.sync_copy(ncb_v, cnt_hbm.at[r, w, 0])


@functools.partial(
    pl.kernel,
    out_type=jax.ShapeDtypeStruct((_N, 128), jnp.float32),
    mesh=_mesh,
    scratch_types=[
        pltpu.VMEM_SHARED((_ACCR, 128), jnp.float32),
        pltpu.VMEM((_CH,), jnp.int32),
        pltpu.VMEM((_CH,), jnp.int32),
        pltpu.VMEM((128,), jnp.int32),
        pltpu.VMEM((_CH, 128), jnp.float32),
    ],
    compiler_params=pltpu.CompilerParams(needs_layout_passes=False),
)
def _agg_kernel(u_hbm, srcl_hbm, dstl_hbm, cnt_hbm, zr_hbm, agg_hbm,
                acc_sh, sidx, didx, cnt_v, rb):
    c = lax.axis_index("c")
    s = lax.axis_index("s")

    for rk in range(_NR // _NC):
        r = c * (_NR // _NC) + rk
        pltpu.sync_copy(zr_hbm, acc_sh.at[pl.ds(s * _STR, _STR)])
        plsc.subcore_barrier()

        for k in range(2):
            f = s * 2 + k
            pltpu.sync_copy(cnt_hbm.at[r, f, 0], cnt_v)
            nch = jnp.max(cnt_v[pl.ds(0, 16)])

            def chunk(j, carry):
                pltpu.sync_copy(srcl_hbm.at[r, f, j], sidx)
                pltpu.sync_copy(dstl_hbm.at[r, f, j], didx)
                pltpu.sync_copy(u_hbm.at[sidx], rb)
                pltpu.sync_copy(rb, acc_sh.at[didx], add=True)
                return carry

            lax.fori_loop(0, nch, chunk, 0)
        plsc.subcore_barrier()

        out_base = r * _RR + s * _STR

        @pl.when(s < _NS - 1)
        def _():
            pltpu.sync_copy(acc_sh.at[pl.ds(s * _STR, _STR)],
                            agg_hbm.at[pl.ds(out_base, _STR)])

        @pl.when((s == _NS - 1) & (r < _NR - 1))
        def _():
            rows = _RR - (_NS - 1) * _STR
            pltpu.sync_copy(acc_sh.at[pl.ds((_NS - 1) * _STR, rows)],
                            agg_hbm.at[pl.ds(out_base, rows)])

        @pl.when((s == _NS - 1) & (r == _NR - 1))
        def _():
            rows = _N - (_NR - 1) * _RR - (_NS - 1) * _STR
            pltpu.sync_copy(acc_sh.at[pl.ds((_NS - 1) * _STR, rows)],
                            agg_hbm.at[pl.ds(out_base, rows)])

        plsc.subcore_barrier()


_BLK = 1000


def _m1_body(x_ref, hist_ref, w1_ref, u1_ref, dis_ref):
    deg = jnp.sum(hist_ref[...], axis=1) + 1.0
    dis = lax.rsqrt(deg)[:, None]
    z = jnp.dot(x_ref[...], w1_ref[...], preferred_element_type=jnp.float32)
    u1_ref[...] = z * dis
    dis_ref[...] = dis


_m1_call = pl.pallas_call(
    _m1_body,
    grid=(_N // _BLK,),
    in_specs=[
        pl.BlockSpec((_BLK, 1024), lambda i: (i, 0)),
        pl.BlockSpec((_BLK, _NW), lambda i: (i, 0)),
        pl.BlockSpec((1024, 128), lambda i: (0, 0)),
    ],
    out_specs=[
        pl.BlockSpec((_BLK, 128), lambda i: (i, 0)),
        pl.BlockSpec((_BLK, 1), lambda i: (i, 0)),
    ],
    out_shape=[
        jax.ShapeDtypeStruct((_N, 128), jnp.float32),
        jax.ShapeDtypeStruct((_N, 1), jnp.float32),
    ],
)


def _m2_body(agg_ref, u_ref, dis_ref, w_ref, b_ref, out_ref):
    dis = dis_ref[...]
    h = jnp.maximum(dis * (agg_ref[...] + u_ref[...]) + b_ref[...], 0.0)
    out_ref[...] = dis * jnp.dot(h, w_ref[...],
                                 preferred_element_type=jnp.float32)


_m2_call = pl.pallas_call(
    _m2_body,
    grid=(_N // _BLK,),
    in_specs=[
        pl.BlockSpec((_BLK, 128), lambda i: (i, 0)),
        pl.BlockSpec((_BLK, 128), lambda i: (i, 0)),
        pl.BlockSpec((_BLK, 1), lambda i: (i, 0)),
        pl.BlockSpec((128, 128), lambda i: (0, 0)),
        pl.BlockSpec((1, 128), lambda i: (0, 0)),
    ],
    out_specs=pl.BlockSpec((_BLK, 128), lambda i: (i, 0)),
    out_shape=jax.ShapeDtypeStruct((_N, 128), jnp.float32),
)


def _m3_body(agg_ref, u_ref, dis_ref, b_ref, out_ref):
    dis = dis_ref[...]
    h = jnp.maximum(dis * (agg_ref[...] + u_ref[...]) + b_ref[...], 0.0)
    out_ref[...] = dis * h


_m3_call = pl.pallas_call(
    _m3_body,
    grid=(_N // _BLK,),
    in_specs=[
        pl.BlockSpec((_BLK, 128), lambda i: (i, 0)),
        pl.BlockSpec((_BLK, 128), lambda i: (i, 0)),
        pl.BlockSpec((_BLK, 1), lambda i: (i, 0)),
        pl.BlockSpec((1, 128), lambda i: (0, 0)),
    ],
    out_specs=pl.BlockSpec((_BLK, 128), lambda i: (i, 0)),
    out_shape=jax.ShapeDtypeStruct((_N, 128), jnp.float32),
)


def _m4_body(agg_ref, u_ref, dis_ref, w3_ref, b3_ref, out_ref):
    dis = dis_ref[...]
    t = dis * (agg_ref[...] + u_ref[...])
    y = jnp.dot(t, w3_ref[...], preferred_element_type=jnp.float32) + b3_ref[...]
    col = lax.broadcasted_iota(jnp.int32, (_BLK, 16), 1)
    y = jnp.where(col < 9, y, -jnp.inf)
    m = jnp.max(y, axis=1, keepdims=True)
    zc = y - m
    ez = jnp.where(col < 9, jnp.exp(zc), 0.0)
    lse = jnp.log(jnp.sum(ez, axis=1, keepdims=True))
    out_ref[...] = (zc - lse)[:, :9]


_m4_call = pl.pallas_call(
    _m4_body,
    grid=(_N // _BLK,),
    in_specs=[
        pl.BlockSpec((_BLK, 128), lambda i: (i, 0)),
        pl.BlockSpec((_BLK, 128), lambda i: (i, 0)),
        pl.BlockSpec((_BLK, 1), lambda i: (i, 0)),
        pl.BlockSpec((128, 16), lambda i: (0, 0)),
        pl.BlockSpec((1, 16), lambda i: (0, 0)),
    ],
    out_specs=pl.BlockSpec((_BLK, 9), lambda i: (i, 0)),
    out_shape=jax.ShapeDtypeStruct((_N, 9), jnp.float32),
)


def kernel(x, edge_index, W1, b1, W2, b2, W3, b3):
    zr = jnp.zeros((_STR, 128), jnp.float32)
    src = edge_index[0]
    dst = edge_index[1]
    hist = _hist_kernel(dst)
    srcl, dstl, cnts = _lists_kernel(src, dst)
    srcl = srcl.reshape(_NR, _NW, _NCH, _CH)
    dstl = dstl.reshape(_NR, _NW, _NCH, _CH)

    u1, dis = _m1_call(x, hist.reshape(_NW, _N).T, W1)
    agg1 = _agg_kernel(u1, srcl, dstl, cnts, zr)
    u2 = _m2_call(agg1, u1, dis, W2, b1.reshape(1, 128))
    agg2 = _agg_kernel(u2, srcl, dstl, cnts, zr)
    u3 = _m3_call(agg2, u2, dis, b2.reshape(1, 128))
    agg3 = _agg_kernel(u3, srcl, dstl, cnts, zr)

    W3p = jnp.pad(W3, ((0, 0), (0, 7)))
    b3p = jnp.pad(b3, (0, 7)).reshape(1, 16)
    return _m4_call(agg3, u3, dis, W3p, b3p)


# speedup vs baseline: 3.5043x; 3.5043x over previous
import functools

import jax
import jax.numpy as jnp
from jax import lax
from jax.experimental import pallas as pl
from jax.experimental.pallas import tpu as pltpu
from jax.experimental.pallas import tpu_sc as plsc

_N = 50000
_E = 800000
_NC, _NS = 2, 16
_NW = _NC * _NS
_ECH = _E // _NW
_NVEC = (_ECH + 15) // 16
_EBUF = _NVEC * 16
_NR = 8
_RR = 6256
_STR = 392
_ACCR = _NS * _STR
_DUMP = 6264
_CH = 128
_NCH = 200
_CAP = _NCH * _CH

_mesh = plsc.VectorSubcoreMesh(core_axis_name="c", subcore_axis_name="s")


def _wid():
    return lax.axis_index("s") * _NC + lax.axis_index("c")


@functools.partial(
    pl.kernel,
    out_type=jax.ShapeDtypeStruct((_NW, 1, _N), jnp.float32),
    mesh=_mesh,
    scratch_types=[
        pltpu.VMEM((_EBUF,), jnp.int32),
        pltpu.VMEM((_N,), jnp.float32),
    ],
    compiler_params=pltpu.CompilerParams(needs_layout_passes=False),
)
def _hist_kernel(d_hbm, hist_hbm, dst_v, hist_v):
    w = _wid()
    base = w * _ECH
    pltpu.sync_copy(d_hbm.at[pl.ds(base, _ECH)], dst_v.at[pl.ds(0, _ECH)])

    def zero(i, carry):
        hist_v[pl.ds(i * 16, 16)] = jnp.zeros((16,), jnp.float32)
        return carry

    lax.fori_loop(0, _N // 16, zero, 0)

    iota = lax.iota(jnp.int32, 16)
    ones = jnp.ones((16,), jnp.float32)

    def body(i, carry):
        d16 = dst_v[pl.ds(i * 16, 16)]
        m = (i * 16 + iota) < _ECH
        plsc.addupdate_scatter(hist_v, [d16], ones, mask=m)
        return carry

    lax.fori_loop(0, _NVEC, body, 0)
    pltpu.sync_copy(hist_v, hist_hbm.at[w, 0])


@functools.partial(
    pl.kernel,
    out_type=(
        jax.ShapeDtypeStruct((_NR, _NW, 1, 2 * _CAP), jnp.int32),
        jax.ShapeDtypeStruct((_NR, _NW, 1, 128), jnp.int32),
    ),
    mesh=_mesh,
    scratch_types=[
        pltpu.VMEM((_EBUF,), jnp.int32),
        pltpu.VMEM((_EBUF,), jnp.int32),
        pltpu.VMEM((2 * _CAP,), jnp.int32),
        pltpu.VMEM((128,), jnp.int32),
    ],
    compiler_params=pltpu.CompilerParams(needs_layout_passes=False),
)
def _lists_kernel(s_hbm, d_hbm, lst_hbm, cnt_hbm,
                  src_v, dst_v, lst_v, ncb_v):
    w = _wid()
    base = w * _ECH
    pltpu.sync_copy(s_hbm.at[pl.ds(base, _ECH)], src_v.at[pl.ds(0, _ECH)])
    pltpu.sync_copy(d_hbm.at[pl.ds(base, _ECH)], dst_v.at[pl.ds(0, _ECH)])
    iota = lax.iota(jnp.int32, 16)
    zs16 = jnp.zeros((16,), jnp.int32)
    dump16 = jnp.full((16,), _DUMP, jnp.int32)

    for r in range(_NR):
        lo = r * _RR

        def body(i, off):
            s16 = src_v[pl.ds(i * 16, 16)]
            d16 = dst_v[pl.ds(i * 16, 16)]
            m = ((i * 16 + iota) < _ECH) & (d16 >= lo) & (d16 < lo + _RR)
            inc = jnp.cumsum(m.astype(jnp.int32))
            pos = off + inc - 1
            pc = ((pos >> 7) << 8) + (pos & 127)
            plsc.store_scatter(lst_v, [pc], s16, mask=m)
            plsc.store_scatter(lst_v, [pc + 128], d16 - lo, mask=m)
            return off + jnp.max(inc)

        off = lax.fori_loop(0, _NVEC, body, jnp.int32(0))

        def fill(t, c):
            p = off + t * 16 + iota
            pc = ((p >> 7) << 8) + (p & 127)
            full = jnp.ones((16,), jnp.bool_)
            plsc.store_scatter(lst_v, [pc], zs16, mask=full)
            plsc.store_scatter(lst_v, [pc + 128], dump16, mask=full)
            return c

        lax.fori_loop(0, _CH // 16, fill, 0)
        nch = (off + _CH - 1) // _CH

        def setnc(i, c):
            ncb_v[pl.ds(i * 16, 16)] = jnp.broadcast_to(nch, (16,)).astype(
                jnp.int32)
            return c

        lax.fori_loop(0, 8, setnc, 0)
        pltpu.sync_copy(lst_v, lst_hbm.at[r, w, 0])
        pltpu.sync_copy(ncb_v, cnt_hbm.at[r, w, 0])


@functools.partial(
    pl.kernel,
    out_type=jax.ShapeDtypeStruct((_N, 128), jnp.float32),
    mesh=_mesh,
    scratch_types=[
        pltpu.VMEM_SHARED((_ACCR, 128), jnp.float32),
        pltpu.VMEM((2, _CH), jnp.int32),
        pltpu.VMEM((128,), jnp.int32),
        pltpu.VMEM((_CH, 128), jnp.float32),
    ],
    compiler_params=pltpu.CompilerParams(needs_layout_passes=False),
)
def _agg_kernel(u_hbm, lst_hbm, cnt_hbm, zr_hbm, agg_hbm,
                acc_sh, cidx, cnt_v, rb):
    c = lax.axis_index("c")
    s = lax.axis_index("s")

    for rk in range(_NR // _NC):
        r = c * (_NR // _NC) + rk
        pltpu.sync_copy(zr_hbm, acc_sh.at[pl.ds(s * _STR, _STR)])
        plsc.subcore_barrier()

        for k in range(2):
            f = s * 2 + k
            pltpu.sync_copy(cnt_hbm.at[r, f, 0], cnt_v)
            nch = jnp.max(cnt_v[pl.ds(0, 16)])

            def chunk(j, carry):
                pltpu.sync_copy(lst_hbm.at[r, f, j], cidx)
                pltpu.sync_copy(u_hbm.at[cidx.at[0]], rb)
                pltpu.sync_copy(rb, acc_sh.at[cidx.at[1]], add=True)
                return carry

            lax.fori_loop(0, nch, chunk, 0)
        plsc.subcore_barrier()

        out_base = r * _RR + s * _STR

        @pl.when(s < _NS - 1)
        def _():
            pltpu.sync_copy(acc_sh.at[pl.ds(s * _STR, _STR)],
                            agg_hbm.at[pl.ds(out_base, _STR)])

        @pl.when((s == _NS - 1) & (r < _NR - 1))
        def _():
            rows = _RR - (_NS - 1) * _STR
            pltpu.sync_copy(acc_sh.at[pl.ds((_NS - 1) * _STR, rows)],
                            agg_hbm.at[pl.ds(out_base, rows)])

        @pl.when((s == _NS - 1) & (r == _NR - 1))
        def _():
            rows = _N - (_NR - 1) * _RR - (_NS - 1) * _STR
            pltpu.sync_copy(acc_sh.at[pl.ds((_NS - 1) * _STR, rows)],
                            agg_hbm.at[pl.ds(out_base, rows)])

        plsc.subcore_barrier()


_BLK = 1000


def _m1_body(x_ref, hist_ref, w1_ref, u1_ref, dis_ref):
    deg = jnp.sum(hist_ref[...], axis=1) + 1.0
    dis = lax.rsqrt(deg)[:, None]
    z = jnp.dot(x_ref[...], w1_ref[...], preferred_element_type=jnp.float32)
    u1_ref[...] = z * dis
    dis_ref[...] = dis


_m1_call = pl.pallas_call(
    _m1_body,
    grid=(_N // _BLK,),
    in_specs=[
        pl.BlockSpec((_BLK, 1024), lambda i: (i, 0)),
        pl.BlockSpec((_BLK, _NW), lambda i: (i, 0)),
        pl.BlockSpec((1024, 128), lambda i: (0, 0)),
    ],
    out_specs=[
        pl.BlockSpec((_BLK, 128), lambda i: (i, 0)),
        pl.BlockSpec((_BLK, 1), lambda i: (i, 0)),
    ],
    out_shape=[
        jax.ShapeDtypeStruct((_N, 128), jnp.float32),
        jax.ShapeDtypeStruct((_N, 1), jnp.float32),
    ],
)


def _m2_body(agg_ref, u_ref, dis_ref, w_ref, b_ref, out_ref):
    dis = dis_ref[...]
    h = jnp.maximum(dis * (agg_ref[...] + u_ref[...]) + b_ref[...], 0.0)
    out_ref[...] = dis * jnp.dot(h, w_ref[...],
                                 preferred_element_type=jnp.float32)


_m2_call = pl.pallas_call(
    _m2_body,
    grid=(_N // _BLK,),
    in_specs=[
        pl.BlockSpec((_BLK, 128), lambda i: (i, 0)),
        pl.BlockSpec((_BLK, 128), lambda i: (i, 0)),
        pl.BlockSpec((_BLK, 1), lambda i: (i, 0)),
        pl.BlockSpec((128, 128), lambda i: (0, 0)),
        pl.BlockSpec((1, 128), lambda i: (0, 0)),
    ],
    out_specs=pl.BlockSpec((_BLK, 128), lambda i: (i, 0)),
    out_shape=jax.ShapeDtypeStruct((_N, 128), jnp.float32),
)


def _m3_body(agg_ref, u_ref, dis_ref, b_ref, out_ref):
    dis = dis_ref[...]
    h = jnp.maximum(dis * (agg_ref[...] + u_ref[...]) + b_ref[...], 0.0)
    out_ref[...] = dis * h


_m3_call = pl.pallas_call(
    _m3_body,
    grid=(_N // _BLK,),
    in_specs=[
        pl.BlockSpec((_BLK, 128), lambda i: (i, 0)),
        pl.BlockSpec((_BLK, 128), lambda i: (i, 0)),
        pl.BlockSpec((_BLK, 1), lambda i: (i, 0)),
        pl.BlockSpec((1, 128), lambda i: (0, 0)),
    ],
    out_specs=pl.BlockSpec((_BLK, 128), lambda i: (i, 0)),
    out_shape=jax.ShapeDtypeStruct((_N, 128), jnp.float32),
)


def _m4_body(agg_ref, u_ref, dis_ref, w3_ref, b3_ref, out_ref):
    dis = dis_ref[...]
    t = dis * (agg_ref[...] + u_ref[...])
    y = jnp.dot(t, w3_ref[...], preferred_element_type=jnp.float32) + b3_ref[...]
    col = lax.broadcasted_iota(jnp.int32, (_BLK, 16), 1)
    y = jnp.where(col < 9, y, -jnp.inf)
    m = jnp.max(y, axis=1, keepdims=True)
    zc = y - m
    ez = jnp.where(col < 9, jnp.exp(zc), 0.0)
    lse = jnp.log(jnp.sum(ez, axis=1, keepdims=True))
    out_ref[...] = (zc - lse)[:, :9]


_m4_call = pl.pallas_call(
    _m4_body,
    grid=(_N // _BLK,),
    in_specs=[
        pl.BlockSpec((_BLK, 128), lambda i: (i, 0)),
        pl.BlockSpec((_BLK, 128), lambda i: (i, 0)),
        pl.BlockSpec((_BLK, 1), lambda i: (i, 0)),
        pl.BlockSpec((128, 16), lambda i: (0, 0)),
        pl.BlockSpec((1, 16), lambda i: (0, 0)),
    ],
    out_specs=pl.BlockSpec((_BLK, 9), lambda i: (i, 0)),
    out_shape=jax.ShapeDtypeStruct((_N, 9), jnp.float32),
)


def kernel(x, edge_index, W1, b1, W2, b2, W3, b3):
    zr = jnp.zeros((_STR, 128), jnp.float32)
    src = edge_index[0]
    dst = edge_index[1]
    hist = _hist_kernel(dst)
    lsts, cnts = _lists_kernel(src, dst)
    lsts = lsts.reshape(_NR, _NW, _NCH, 2, _CH)

    u1, dis = _m1_call(x, hist.reshape(_NW, _N).T, W1)
    agg1 = _agg_kernel(u1, lsts, cnts, zr)
    u2 = _m2_call(agg1, u1, dis, W2, b1.reshape(1, 128))
    agg2 = _agg_kernel(u2, lsts, cnts, zr)
    u3 = _m3_call(agg2, u2, dis, b2.reshape(1, 128))
    agg3 = _agg_kernel(u3, lsts, cnts, zr)

    W3p = jnp.pad(W3, ((0, 0), (0, 7)))
    b3p = jnp.pad(b3, (0, 7)).reshape(1, 16)
    return _m4_call(agg3, u3, dis, W3p, b3p)


# speedup vs baseline: 4.6964x; 1.3402x over previous
import functools

import jax
import jax.numpy as jnp
from jax import lax
from jax.experimental import pallas as pl
from jax.experimental.pallas import tpu as pltpu
from jax.experimental.pallas import tpu_sc as plsc

_N = 50000
_E = 800000
_NC, _NS = 2, 16
_NW = _NC * _NS
_ECH = _E // _NW
_NVEC = (_ECH + 15) // 16
_EBUF = _NVEC * 16
_NR = 4
_RR = 12504
_STR = 784
_ACCR = _NS * _STR
_DUMP = 12520
_CH = 128
_NCH = 200
_CAP = _NCH * _CH

_mesh = plsc.VectorSubcoreMesh(core_axis_name="c", subcore_axis_name="s")


def _wid():
    return lax.axis_index("s") * _NC + lax.axis_index("c")


@functools.partial(
    pl.kernel,
    out_type=jax.ShapeDtypeStruct((_NW, 1, _N), jnp.float32),
    mesh=_mesh,
    scratch_types=[
        pltpu.VMEM((_EBUF,), jnp.int32),
        pltpu.VMEM((_N,), jnp.float32),
    ],
    compiler_params=pltpu.CompilerParams(needs_layout_passes=False),
)
def _hist_kernel(d_hbm, hist_hbm, dst_v, hist_v):
    w = _wid()
    base = w * _ECH
    pltpu.sync_copy(d_hbm.at[pl.ds(base, _ECH)], dst_v.at[pl.ds(0, _ECH)])

    def zero(i, carry):
        hist_v[pl.ds(i * 16, 16)] = jnp.zeros((16,), jnp.float32)
        return carry

    lax.fori_loop(0, _N // 16, zero, 0)

    iota = lax.iota(jnp.int32, 16)
    ones = jnp.ones((16,), jnp.float32)

    def body(i, carry):
        d16 = dst_v[pl.ds(i * 16, 16)]
        m = (i * 16 + iota) < _ECH
        plsc.addupdate_scatter(hist_v, [d16], ones, mask=m)
        return carry

    lax.fori_loop(0, _NVEC, body, 0)
    pltpu.sync_copy(hist_v, hist_hbm.at[w, 0])


@functools.partial(
    pl.kernel,
    out_type=(
        jax.ShapeDtypeStruct((_NR, _NW, 1, 2 * _CAP), jnp.int32),
        jax.ShapeDtypeStruct((_NR, _NW, 1, 128), jnp.int32),
    ),
    mesh=_mesh,
    scratch_types=[
        pltpu.VMEM((_EBUF,), jnp.int32),
        pltpu.VMEM((_EBUF,), jnp.int32),
        pltpu.VMEM((2 * _CAP,), jnp.int32),
        pltpu.VMEM((128,), jnp.int32),
    ],
    compiler_params=pltpu.CompilerParams(needs_layout_passes=False),
)
def _lists_kernel(s_hbm, d_hbm, lst_hbm, cnt_hbm,
                  src_v, dst_v, lst_v, ncb_v):
    w = _wid()
    base = w * _ECH
    pltpu.sync_copy(s_hbm.at[pl.ds(base, _ECH)], src_v.at[pl.ds(0, _ECH)])
    pltpu.sync_copy(d_hbm.at[pl.ds(base, _ECH)], dst_v.at[pl.ds(0, _ECH)])
    iota = lax.iota(jnp.int32, 16)
    zs16 = jnp.zeros((16,), jnp.int32)
    dump16 = jnp.full((16,), _DUMP, jnp.int32)

    for r in range(_NR):
        lo = r * _RR

        def body(i, off):
            s16 = src_v[pl.ds(i * 16, 16)]
            d16 = dst_v[pl.ds(i * 16, 16)]
            m = ((i * 16 + iota) < _ECH) & (d16 >= lo) & (d16 < lo + _RR)
            inc = jnp.cumsum(m.astype(jnp.int32))
            pos = off + inc - 1
            pc = ((pos >> 7) << 8) + (pos & 127)
            plsc.store_scatter(lst_v, [pc], s16, mask=m)
            plsc.store_scatter(lst_v, [pc + 128], d16 - lo, mask=m)
            return off + jnp.max(inc)

        off = lax.fori_loop(0, _NVEC, body, jnp.int32(0))

        def fill(t, c):
            p = off + t * 16 + iota
            pc = ((p >> 7) << 8) + (p & 127)
            full = jnp.ones((16,), jnp.bool_)
            plsc.store_scatter(lst_v, [pc], zs16, mask=full)
            plsc.store_scatter(lst_v, [pc + 128], dump16, mask=full)
            return c

        lax.fori_loop(0, _CH // 16, fill, 0)
        nch = (off + _CH - 1) // _CH

        def setnc(i, c):
            ncb_v[pl.ds(i * 16, 16)] = jnp.broadcast_to(nch, (16,)).astype(
                jnp.int32)
            return c

        lax.fori_loop(0, 8, setnc, 0)
        pltpu.sync_copy(lst_v, lst_hbm.at[r, w, 0])
        pltpu.sync_copy(ncb_v, cnt_hbm.at[r, w, 0])


@functools.partial(
    pl.kernel,
    out_type=jax.ShapeDtypeStruct((_N, 128), jnp.float32),
    mesh=_mesh,
    scratch_types=[
        pltpu.VMEM_SHARED((_ACCR, 128), jnp.float32),
        pltpu.VMEM((2, _CH), jnp.int32),
        pltpu.VMEM((128,), jnp.int32),
        pltpu.VMEM((_CH, 128), jnp.float32),
    ],
    compiler_params=pltpu.CompilerParams(needs_layout_passes=False),
)
def _agg_kernel(u_hbm, lst_hbm, cnt_hbm, zr_hbm, agg_hbm,
                acc_sh, cidx, cnt_v, rb):
    c = lax.axis_index("c")
    s = lax.axis_index("s")

    for rk in range(_NR // _NC):
        r = c * (_NR // _NC) + rk
        pltpu.sync_copy(zr_hbm, acc_sh.at[pl.ds(s * _STR, _STR)])
        plsc.subcore_barrier()

        for k in range(2):
            f = s * 2 + k
            pltpu.sync_copy(cnt_hbm.at[r, f, 0], cnt_v)
            nch = jnp.max(cnt_v[pl.ds(0, 16)])

            def chunk(j, carry):
                pltpu.sync_copy(lst_hbm.at[r, f, j], cidx)
                pltpu.sync_copy(u_hbm.at[cidx.at[0]], rb)
                pltpu.sync_copy(rb, acc_sh.at[cidx.at[1]], add=True)
                return carry

            lax.fori_loop(0, nch, chunk, 0)
        plsc.subcore_barrier()

        out_base = r * _RR + s * _STR

        @pl.when(s < _NS - 1)
        def _():
            pltpu.sync_copy(acc_sh.at[pl.ds(s * _STR, _STR)],
                            agg_hbm.at[pl.ds(out_base, _STR)])

        @pl.when((s == _NS - 1) & (r < _NR - 1))
        def _():
            rows = _RR - (_NS - 1) * _STR
            pltpu.sync_copy(acc_sh.at[pl.ds((_NS - 1) * _STR, rows)],
                            agg_hbm.at[pl.ds(out_base, rows)])

        @pl.when((s == _NS - 1) & (r == _NR - 1))
        def _():
            rows = _N - (_NR - 1) * _RR - (_NS - 1) * _STR
            pltpu.sync_copy(acc_sh.at[pl.ds((_NS - 1) * _STR, rows)],
                            agg_hbm.at[pl.ds(out_base, rows)])

        plsc.subcore_barrier()


_BLK = 1000


def _m1_body(x_ref, hist_ref, w1_ref, u1_ref, dis_ref):
    deg = jnp.sum(hist_ref[...], axis=1) + 1.0
    dis = lax.rsqrt(deg)[:, None]
    z = jnp.dot(x_ref[...], w1_ref[...], preferred_element_type=jnp.float32)
    u1_ref[...] = z * dis
    dis_ref[...] = dis


_m1_call = pl.pallas_call(
    _m1_body,
    grid=(_N // _BLK,),
    in_specs=[
        pl.BlockSpec((_BLK, 1024), lambda i: (i, 0)),
        pl.BlockSpec((_BLK, _NW), lambda i: (i, 0)),
        pl.BlockSpec((1024, 128), lambda i: (0, 0)),
    ],
    out_specs=[
        pl.BlockSpec((_BLK, 128), lambda i: (i, 0)),
        pl.BlockSpec((_BLK, 1), lambda i: (i, 0)),
    ],
    out_shape=[
        jax.ShapeDtypeStruct((_N, 128), jnp.float32),
        jax.ShapeDtypeStruct((_N, 1), jnp.float32),
    ],
)


def _m2_body(agg_ref, u_ref, dis_ref, w_ref, b_ref, out_ref):
    dis = dis_ref[...]
    h = jnp.maximum(dis * (agg_ref[...] + u_ref[...]) + b_ref[...], 0.0)
    out_ref[...] = dis * jnp.dot(h, w_ref[...],
                                 preferred_element_type=jnp.float32)


_m2_call = pl.pallas_call(
    _m2_body,
    grid=(_N // _BLK,),
    in_specs=[
        pl.BlockSpec((_BLK, 128), lambda i: (i, 0)),
        pl.BlockSpec((_BLK, 128), lambda i: (i, 0)),
        pl.BlockSpec((_BLK, 1), lambda i: (i, 0)),
        pl.BlockSpec((128, 128), lambda i: (0, 0)),
        pl.BlockSpec((1, 128), lambda i: (0, 0)),
    ],
    out_specs=pl.BlockSpec((_BLK, 128), lambda i: (i, 0)),
    out_shape=jax.ShapeDtypeStruct((_N, 128), jnp.float32),
)


def _m3_body(agg_ref, u_ref, dis_ref, b_ref, out_ref):
    dis = dis_ref[...]
    h = jnp.maximum(dis * (agg_ref[...] + u_ref[...]) + b_ref[...], 0.0)
    out_ref[...] = dis * h


_m3_call = pl.pallas_call(
    _m3_body,
    grid=(_N // _BLK,),
    in_specs=[
        pl.BlockSpec((_BLK, 128), lambda i: (i, 0)),
        pl.BlockSpec((_BLK, 128), lambda i: (i, 0)),
        pl.BlockSpec((_BLK, 1), lambda i: (i, 0)),
        pl.BlockSpec((1, 128), lambda i: (0, 0)),
    ],
    out_specs=pl.BlockSpec((_BLK, 128), lambda i: (i, 0)),
    out_shape=jax.ShapeDtypeStruct((_N, 128), jnp.float32),
)


def _m4_body(agg_ref, u_ref, dis_ref, w3_ref, b3_ref, out_ref):
    dis = dis_ref[...]
    t = dis * (agg_ref[...] + u_ref[...])
    y = jnp.dot(t, w3_ref[...], preferred_element_type=jnp.float32) + b3_ref[...]
    col = lax.broadcasted_iota(jnp.int32, (_BLK, 16), 1)
    y = jnp.where(col < 9, y, -jnp.inf)
    m = jnp.max(y, axis=1, keepdims=True)
    zc = y - m
    ez = jnp.where(col < 9, jnp.exp(zc), 0.0)
    lse = jnp.log(jnp.sum(ez, axis=1, keepdims=True))
    out_ref[...] = (zc - lse)[:, :9]


_m4_call = pl.pallas_call(
    _m4_body,
    grid=(_N // _BLK,),
    in_specs=[
        pl.BlockSpec((_BLK, 128), lambda i: (i, 0)),
        pl.BlockSpec((_BLK, 128), lambda i: (i, 0)),
        pl.BlockSpec((_BLK, 1), lambda i: (i, 0)),
        pl.BlockSpec((128, 16), lambda i: (0, 0)),
        pl.BlockSpec((1, 16), lambda i: (0, 0)),
    ],
    out_specs=pl.BlockSpec((_BLK, 9), lambda i: (i, 0)),
    out_shape=jax.ShapeDtypeStruct((_N, 9), jnp.float32),
)


def kernel(x, edge_index, W1, b1, W2, b2, W3, b3):
    zr = jnp.zeros((_STR, 128), jnp.float32)
    src = edge_index[0]
    dst = edge_index[1]
    hist = _hist_kernel(dst)
    lsts, cnts = _lists_kernel(src, dst)
    lsts = lsts.reshape(_NR, _NW, _NCH, 2, _CH)

    u1, dis = _m1_call(x, hist.reshape(_NW, _N).T, W1)
    agg1 = _agg_kernel(u1, lsts, cnts, zr)
    u2 = _m2_call(agg1, u1, dis, W2, b1.reshape(1, 128))
    agg2 = _agg_kernel(u2, lsts, cnts, zr)
    u3 = _m3_call(agg2, u2, dis, b2.reshape(1, 128))
    agg3 = _agg_kernel(u3, lsts, cnts, zr)

    W3p = jnp.pad(W3, ((0, 0), (0, 7)))
    b3p = jnp.pad(b3, (0, 7)).reshape(1, 16)
    return _m4_call(agg3, u3, dis, W3p, b3p)
